# Initial kernel scaffold; baseline (speedup 1.0000x reference)
import jax
import jax.numpy as jnp
from jax.experimental import pallas as pl


def kernel(ray_coordinate_ref, feat_volume):
    raise NotImplementedError("write your pallas kernel here")


# speedup vs baseline: 3.4940x; 3.4940x over previous
import functools

import jax
import jax.numpy as jnp
from jax import lax
from jax.experimental import pallas as pl
from jax.experimental.pallas import tpu as pltpu
from jax.experimental.pallas import tpu_sc as plsc

_NC = 2
_NS = 16
_NW = _NC * _NS
_L = 16

_CH = 128


def _trilinear_sc(coords, table, *, n_pts, d, hv, wv, c):
    np_per_w = n_pts // _NW
    nch = np_per_w // _CH
    assert nch % 2 == 0
    mesh = plsc.VectorSubcoreMesh(
        core_axis_name="c", subcore_axis_name="s", num_cores=_NC,
        num_subcores=_NS)

    @functools.partial(
        pl.kernel,
        out_type=jax.ShapeDtypeStruct((n_pts, c), jnp.float32),
        mesh=mesh,
        compiler_params=pltpu.CompilerParams(
            needs_layout_passes=False, use_tc_tiling_on_sc=False),
        scratch_types=[
            pltpu.VMEM((2, 3, _CH), jnp.float32),
            pltpu.VMEM((2, 8, _CH), jnp.int32),
            pltpu.VMEM((2, 8, _CH), jnp.float32),
            pltpu.VMEM((2, 8 * _CH, 16), jnp.float32),
            pltpu.VMEM((2, _CH, 16), jnp.float32),
            pltpu.SemaphoreType.DMA,
            pltpu.SemaphoreType.DMA,
            pltpu.SemaphoreType.DMA,
            pltpu.SemaphoreType.DMA,
            pltpu.SemaphoreType.DMA,
            pltpu.SemaphoreType.DMA,
        ],
    )
    def k(coords_hbm, table_hbm, out_hbm,
          cbuf, idxbuf, wbuf, rbuf, obuf,
          sem_c0, sem_c1, sem_g0, sem_g1, sem_o0, sem_o1):
        wid = lax.axis_index("s") * _NC + lax.axis_index("c")
        base0 = wid * np_per_w
        sem_c = (sem_c0, sem_c1)
        sem_g = (sem_g0, sem_g1)
        sem_o = (sem_o0, sem_o1)
        lanes = lax.iota(jnp.int32, _L)

        def fire_coords(g, s):
            base = base0 + g * _CH
            pltpu.async_copy(
                coords_hbm.at[:, pl.ds(base, _CH)], cbuf.at[s], sem_c[s])

        def wait_coords(g, s):
            base = base0 + g * _CH
            pltpu.make_async_copy(
                coords_hbm.at[:, pl.ds(base, _CH)], cbuf.at[s],
                sem_c[s]).wait()

        def compute_idx(g, s):
            for i in range(_CH // _L):
                sl = pl.ds(i * _L, _L)
                x = cbuf[s, 0, sl]
                y = cbuf[s, 1, sl]
                z = cbuf[s, 2, sl]
                ix = (x * 2.0 - 1.0 + 1.0) * 0.5 * float(wv - 1)
                iy = (y * 2.0 - 1.0 + 1.0) * 0.5 * float(hv - 1)
                iz = (z * 2.0 - 1.0 + 1.0) * 0.5 * float(d - 1)
                x0 = ix.astype(jnp.int32)
                y0 = iy.astype(jnp.int32)
                z0 = iz.astype(jnp.int32)
                wx1 = ix - x0.astype(jnp.float32)
                wy1 = iy - y0.astype(jnp.float32)
                wz1 = iz - z0.astype(jnp.float32)
                wx0 = 1.0 - wx1
                wy0 = 1.0 - wy1
                wz0 = 1.0 - wz1
                xc0 = jnp.minimum(jnp.maximum(x0, 0), wv - 1)
                yc0 = jnp.minimum(jnp.maximum(y0, 0), hv - 1)
                zc0 = jnp.minimum(jnp.maximum(z0, 0), d - 1)
                xc1 = jnp.minimum(x0 + 1, wv - 1)
                yc1 = jnp.minimum(y0 + 1, hv - 1)
                zc1 = jnp.minimum(z0 + 1, d - 1)
                ty0 = yc0 * wv
                ty1 = yc1 * wv
                tz0 = zc0 * (hv * wv)
                tz1 = zc1 * (hv * wv)
                idxbuf[s, 0, sl] = tz0 + ty0 + xc0
                idxbuf[s, 1, sl] = tz0 + ty0 + xc1
                idxbuf[s, 2, sl] = tz0 + ty1 + xc0
                idxbuf[s, 3, sl] = tz0 + ty1 + xc1
                idxbuf[s, 4, sl] = tz1 + ty0 + xc0
                idxbuf[s, 5, sl] = tz1 + ty0 + xc1
                idxbuf[s, 6, sl] = tz1 + ty1 + xc0
                idxbuf[s, 7, sl] = tz1 + ty1 + xc1
                wzy00 = wz0 * wy0
                wzy01 = wz0 * wy1
                wzy10 = wz1 * wy0
                wzy11 = wz1 * wy1
                wbuf[s, 0, sl] = wzy00 * wx0
                wbuf[s, 1, sl] = wzy00 * wx1
                wbuf[s, 2, sl] = wzy01 * wx0
                wbuf[s, 3, sl] = wzy01 * wx1
                wbuf[s, 4, sl] = wzy10 * wx0
                wbuf[s, 5, sl] = wzy10 * wx1
                wbuf[s, 6, sl] = wzy11 * wx0
                wbuf[s, 7, sl] = wzy11 * wx1

        def fire_gathers(s):
            for kk in range(8):
                pltpu.async_copy(
                    table_hbm.at[idxbuf.at[s, kk]],
                    rbuf.at[s, pl.ds(kk * _CH, _CH)], sem_g[s])

        def wait_gathers(s):
            for kk in range(8):
                pltpu.make_async_copy(
                    table_hbm.at[idxbuf.at[s, kk]],
                    rbuf.at[s, pl.ds(kk * _CH, _CH)], sem_g[s]).wait()

        def combine(s):
            def pgroup(i, _):
                sl = pl.ds(i * _L, _L)
                wv8 = [wbuf[s, kk, sl] for kk in range(8)]
                pvec = i * _L + lanes
                for cc in range(16):
                    ccv = jnp.full((_L,), cc, jnp.int32)
                    acc = wv8[0] * plsc.load_gather(
                        rbuf, [jnp.full((_L,), s, jnp.int32), pvec, ccv])
                    for kk in range(1, 8):
                        vals = plsc.load_gather(
                            rbuf,
                            [jnp.full((_L,), s, jnp.int32),
                             kk * _CH + pvec, ccv])
                        acc = acc + wv8[kk] * vals
                    plsc.store_scatter(
                        obuf, [jnp.full((_L,), s, jnp.int32), pvec, ccv], acc)
                return 0

            lax.fori_loop(0, _CH // _L, pgroup, 0)

        def fire_out(g, s):
            base = base0 + g * _CH
            pltpu.async_copy(
                obuf.at[s], out_hbm.at[pl.ds(base, _CH)], sem_o[s])

        def wait_out(g, s):
            base = base0 + g * _CH
            pltpu.make_async_copy(
                obuf.at[s], out_hbm.at[pl.ds(base, _CH)], sem_o[s]).wait()

        fire_coords(0, 0)
        fire_coords(1, 1)
        wait_coords(0, 0)
        compute_idx(0, 0)
        fire_gathers(0)
        fire_coords(2, 0)

        def step(i, _):
            g = 2 * i
            h = g + 1
            wait_coords(h, 1)
            compute_idx(h, 1)
            fire_gathers(1)

            @pl.when(h + 2 < nch)
            def _():
                fire_coords(h + 2, 1)

            wait_gathers(0)

            @pl.when(g >= 2)
            def _():
                wait_out(g - 2, 0)

            combine(0)
            fire_out(g, 0)

            g1 = 2 * i + 1
            h1 = g1 + 1

            @pl.when(h1 < nch)
            def _():
                wait_coords(h1, 0)
                compute_idx(h1, 0)
                fire_gathers(0)

                @pl.when(h1 + 2 < nch)
                def _():
                    fire_coords(h1 + 2, 0)

            wait_gathers(1)

            @pl.when(g1 >= 2)
            def _():
                wait_out(g1 - 2, 1)

            combine(1)
            fire_out(g1, 1)
            return 0

        lax.fori_loop(0, nch // 2, step, 0)
        wait_out(nch - 2, 0)
        wait_out(nch - 1, 1)

    return k(coords, table)


def kernel(ray_coordinate_ref, feat_volume):
    h = ray_coordinate_ref.shape[-3]
    w = ray_coordinate_ref.shape[-2]
    b, c, d, hv, wv = feat_volume.shape
    n = h * w
    coords = jnp.transpose(ray_coordinate_ref.reshape(n, 3))
    table = jnp.transpose(feat_volume[0].reshape(c, d * hv * wv))
    out = _trilinear_sc(coords, table, n_pts=n, d=d, hv=hv, wv=wv, c=c)
    return out.reshape(h, w, c)
